# baseline (speedup 1.0000x reference)
import functools

import jax
import jax.numpy as jnp
from jax import lax
from jax.experimental import pallas as pl
from jax.experimental.pallas import tpu as pltpu
from jax.experimental.pallas import tpu_sc as plsc

from jax._src.pallas.mosaic import lowering as _tc_lowering

_ir = _tc_lowering.ir
_tpu = _tc_lowering.tpu


def _reshape_memref_as_reinterpret(ref, reshaper, ref_aval, ref_block_shape):
    ref_ty = _ir.MemRefType(ref.type)
    if len(reshaper.shape) == 1:
        tile, stride = "1", "1"
    else:
        tile = ",".join(["1"] * (len(reshaper.shape) - 1) + [str(reshaper.shape[-1])])
        stride = ",".join(["1"] * len(reshaper.shape))
    layout = _ir.Attribute.parse(f"#tpu.tiled<({tile}),[{stride}]>")
    target = _ir.MemRefType.get(
        reshaper.shape, ref_ty.element_type, layout, ref_ty.memory_space
    )
    return _tpu.reinterpret_cast(target, ref), reshaper.shape


_tc_lowering._reshape_memref = _reshape_memref_as_reinterpret

_orig_slice_memref = _tc_lowering._slice_memref


def _slice_memref_keep_layout(ref, indexer, ref_aval, ref_block_shape):
    ref_ty = _ir.MemRefType(ref.type)
    if "#tpu.tiled<(1" not in str(ref_ty):
        return _orig_slice_memref(ref, indexer, ref_aval, ref_block_shape)
    starts, sizes, strides, squeeze_dims, ref_block_shape = (
        _tc_lowering._indexer_to_start_size_stride(
            indexer, ref_block_shape, cast_to_index=False
        )
    )
    assert all((s is None or s == 1) for s in strides)
    assert not any(squeeze_dims)
    static_sizes = []
    dynamic_sizes = []
    for s in sizes:
        if not isinstance(s, _ir.Value):
            static_sizes.append(s)
        elif (v := _tc_lowering._fold_and_get_constant_value(s)) is not None:
            static_sizes.append(v)
        else:
            static_sizes.append(_ir.ShapedType.get_dynamic_size())
            dynamic_sizes.append(s)
    out_ty = _ir.MemRefType.get(
        static_sizes, ref_ty.element_type, ref_ty.layout, ref_ty.memory_space
    )
    return tuple(
        (_tpu.memref_slice(out_ty, ref, starts, dynamic_sizes), ref_block_shape)
    )


_tc_lowering._slice_memref = _slice_memref_keep_layout


def kernel(message, weight):
    (B,) = message.shape
    V, D = weight.shape
    info = plsc.get_sparse_core_info()
    nc, ns, L = info.num_cores, info.num_subcores, info.num_lanes
    nw = nc * ns
    bw = B // nw
    W = bw * D

    t_tiles = (V + 127) // 128
    v_full = (V // 128) * 128
    flat_words = D * V
    s_stride = t_tiles * 1024

    mesh = plsc.VectorSubcoreMesh(core_axis_name="c", subcore_axis_name="s")

    @functools.partial(
        pl.kernel,
        mesh=mesh,
        out_type=jax.ShapeDtypeStruct((D, B), jnp.float32),
        compiler_params=pltpu.CompilerParams(needs_layout_passes=False),
        scratch_types=[
            pltpu.VMEM((bw,), jnp.int32),
            pltpu.VMEM((W,), jnp.int32),
            pltpu.VMEM((D, bw), jnp.float32),
            pltpu.VMEM((256, 128), jnp.float32),
            pltpu.VMEM((8, V - v_full), jnp.float32),
            pltpu.SemaphoreType.DMA,
        ],
    )
    def gather_rows(
        idx_hbm, tab_hbm, out_hbm, bidx_v, addr_v, rows_v, rows4_v, tail_v, sem
    ):
        wid = lax.axis_index("s") * nc + lax.axis_index("c")
        base = wid * bw
        pltpu.sync_copy(idx_hbm.at[pl.ds(base, bw)], bidx_v)

        lanes = lax.iota(jnp.int32, L)
        n_t4 = bw // 128

        @pl.loop(0, bw // L, init_carry=jnp.int32(0), unroll=1)
        def build(q, maxv):
            vec = bidx_v[pl.ds(q * L, L)]
            addr0 = (vec >> 7) * 1024 + (vec & 127)
            for s in range(2):
                for r in range(8):
                    a = addr0 + (s * s_stride + r * 128)
                    if s == 1:
                        a = jnp.where(a >= flat_words, 0, a)
                    addr_v[pl.ds((8 * s + r) * bw + q * L, L)] = a
            return jnp.maximum(maxv, jnp.max(vec))

        pltpu.async_copy(
            tab_hbm.reshape(flat_words, 1).at[addr_v],
            rows4_v.at[pl.ds(0, 64), :].reshape(W, 1),
            sem,
        ).wait()

        @pl.loop(0, bw // L, unroll=1)
        def compact(k):
            col = (k & 7) * L + lanes
            rowbase = k >> 3
            zero = jnp.zeros((L,), jnp.int32)
            for d in range(D):
                row = zero + (rowbase + d * 4)
                val = plsc.load_gather(rows4_v, [row, col])
                rows_v[d, pl.ds(k * L, L)] = val

        @pl.when(build >= v_full)
        def _tail():
            pltpu.sync_copy(
                tab_hbm.at[pl.ds(8, 8), pl.ds(v_full, V - v_full)], tail_v
            )

            @pl.loop(0, bw // L, unroll=1)
            def patch(q):
                vec = bidx_v[pl.ds(q * L, L)]
                m = vec >= v_full
                col = jnp.where(m, vec - v_full, 0)
                j = q * L + lanes
                for r in range(8):
                    val = plsc.load_gather(
                        tail_v, [jnp.full((L,), r, jnp.int32), col], mask=m
                    )
                    plsc.store_scatter(
                        rows_v, [jnp.full((L,), 8 + r, jnp.int32), j], val, mask=m
                    )

        pltpu.sync_copy(rows_v, out_hbm.at[:, pl.ds(base, bw)])

    return gather_rows(message.astype(jnp.int32), weight.T).T


# speedup vs baseline: 1.0827x; 1.0827x over previous
import functools

import jax
import jax.numpy as jnp
from jax import lax
from jax.experimental import pallas as pl
from jax.experimental.pallas import tpu as pltpu
from jax.experimental.pallas import tpu_sc as plsc

from jax._src.pallas.mosaic import lowering as _tc_lowering

_ir = _tc_lowering.ir
_tpu = _tc_lowering.tpu


def _reshape_memref_as_reinterpret(ref, reshaper, ref_aval, ref_block_shape):
    ref_ty = _ir.MemRefType(ref.type)
    if len(reshaper.shape) == 1:
        tile, stride = "1", "1"
    elif (
        len(reshaper.shape) == 2
        and reshaper.shape[-1] % 128 == 0
        and reshaper.shape[-2] % 8 == 0
    ):
        tile = "8,128"
        stride = f"{reshaper.shape[-1] // 128},1"
    else:
        tile = ",".join(["1"] * (len(reshaper.shape) - 1) + [str(reshaper.shape[-1])])
        stride = ",".join(["1"] * len(reshaper.shape))
    layout = _ir.Attribute.parse(f"#tpu.tiled<({tile}),[{stride}]>")
    target = _ir.MemRefType.get(
        reshaper.shape, ref_ty.element_type, layout, ref_ty.memory_space
    )
    return _tpu.reinterpret_cast(target, ref), reshaper.shape


_tc_lowering._reshape_memref = _reshape_memref_as_reinterpret

_orig_slice_memref = _tc_lowering._slice_memref


def _slice_memref_keep_layout(ref, indexer, ref_aval, ref_block_shape):
    ref_ty = _ir.MemRefType(ref.type)
    if "#tpu.tiled<(1" not in str(ref_ty):
        return _orig_slice_memref(ref, indexer, ref_aval, ref_block_shape)
    starts, sizes, strides, squeeze_dims, ref_block_shape = (
        _tc_lowering._indexer_to_start_size_stride(
            indexer, ref_block_shape, cast_to_index=False
        )
    )
    assert all((s is None or s == 1) for s in strides)
    assert not any(squeeze_dims)
    static_sizes = []
    dynamic_sizes = []
    for s in sizes:
        if not isinstance(s, _ir.Value):
            static_sizes.append(s)
        elif (v := _tc_lowering._fold_and_get_constant_value(s)) is not None:
            static_sizes.append(v)
        else:
            static_sizes.append(_ir.ShapedType.get_dynamic_size())
            dynamic_sizes.append(s)
    out_ty = _ir.MemRefType.get(
        static_sizes, ref_ty.element_type, ref_ty.layout, ref_ty.memory_space
    )
    return tuple(
        (_tpu.memref_slice(out_ty, ref, starts, dynamic_sizes), ref_block_shape)
    )


_tc_lowering._slice_memref = _slice_memref_keep_layout


def kernel(message, weight):
    (B,) = message.shape
    V, D = weight.shape
    info = plsc.get_sparse_core_info()
    nc, ns, L = info.num_cores, info.num_subcores, info.num_lanes
    nw = nc * ns
    bw = B // nw
    W = bw * D

    t_tiles = (V + 127) // 128
    v_full = (V // 128) * 128
    flat_words = D * V
    s_stride = t_tiles * 1024

    mesh = plsc.VectorSubcoreMesh(core_axis_name="c", subcore_axis_name="s")

    @functools.partial(
        pl.kernel,
        mesh=mesh,
        out_type=jax.ShapeDtypeStruct((D, B), jnp.float32),
        compiler_params=pltpu.CompilerParams(needs_layout_passes=False),
        scratch_types=[
            pltpu.VMEM((bw,), jnp.int32),
            pltpu.VMEM((W,), jnp.int32),
            pltpu.VMEM((D, bw), jnp.float32),
            pltpu.VMEM((256, 128), jnp.float32),
            pltpu.VMEM((8, V - v_full), jnp.float32),
            pltpu.SemaphoreType.DMA,
        ],
    )
    def gather_rows(
        idx_hbm, tab_hbm, out_hbm, bidx_v, addr_v, rows_v, rows4_v, tail_v, sem
    ):
        wid = lax.axis_index("s") * nc + lax.axis_index("c")
        base = wid * bw
        pltpu.sync_copy(idx_hbm.at[pl.ds(base, bw)], bidx_v)

        lanes = lax.iota(jnp.int32, L)
        n_t4 = bw // 128

        @pl.loop(0, bw // L, init_carry=jnp.int32(0), unroll=1)
        def build(q, maxv):
            vec = bidx_v[pl.ds(q * L, L)]
            addr0 = (vec >> 7) * 1024 + (vec & 127)
            qpos = (q >> 3) * 1024 + (q & 7) * L
            for s in range(2):
                for r in range(8):
                    a = addr0 + (s * s_stride + r * 128)
                    if s == 1:
                        a = jnp.where(a >= flat_words, 0, a)
                    addr_v[pl.ds(s * 4096 + r * 128 + qpos, L)] = a
            return jnp.maximum(maxv, jnp.max(vec))

        pltpu.async_copy(
            tab_hbm.reshape(flat_words, 1).at[addr_v],
            rows4_v.at[pl.ds(0, 64), :].reshape(W, 1),
            sem,
        ).wait()


        @pl.when(build >= v_full)
        def _tail():
            pltpu.sync_copy(
                tab_hbm.at[pl.ds(8, 8), pl.ds(v_full, V - v_full)], tail_v
            )

            @pl.loop(0, bw // L, unroll=1)
            def patch(q):
                vec = bidx_v[pl.ds(q * L, L)]
                m = vec >= v_full
                col = jnp.where(m, vec - v_full, 0)
                qpos = 4096 + (q >> 3) * 1024 + (q & 7) * L + lanes
                for r in range(8):
                    val = plsc.load_gather(
                        tail_v, [jnp.full((L,), r, jnp.int32), col], mask=m
                    )
                    p = qpos + r * 128
                    plsc.store_scatter(rows4_v, [p >> 7, p & 127], val, mask=m)

        pltpu.sync_copy(
            rows4_v.at[pl.ds(0, 64), :].reshape(D, bw),
            out_hbm.at[:, pl.ds(base, bw)],
        )

    return gather_rows(message.astype(jnp.int32), weight.T).T


# speedup vs baseline: 1.0886x; 1.0055x over previous
import functools

import jax
import jax.numpy as jnp
from jax import lax
from jax.experimental import pallas as pl
from jax.experimental.pallas import tpu as pltpu
from jax.experimental.pallas import tpu_sc as plsc

from jax._src.pallas.mosaic import lowering as _tc_lowering

_ir = _tc_lowering.ir
_tpu = _tc_lowering.tpu


def _reshape_memref_as_reinterpret(ref, reshaper, ref_aval, ref_block_shape):
    ref_ty = _ir.MemRefType(ref.type)
    if len(reshaper.shape) == 1:
        tile, stride = "1", "1"
    elif (
        len(reshaper.shape) == 2
        and reshaper.shape[-1] % 128 == 0
        and reshaper.shape[-2] % 8 == 0
    ):
        tile = "8,128"
        stride = f"{reshaper.shape[-1] // 128},1"
    else:
        tile = ",".join(["1"] * (len(reshaper.shape) - 1) + [str(reshaper.shape[-1])])
        stride = ",".join(["1"] * len(reshaper.shape))
    layout = _ir.Attribute.parse(f"#tpu.tiled<({tile}),[{stride}]>")
    target = _ir.MemRefType.get(
        reshaper.shape, ref_ty.element_type, layout, ref_ty.memory_space
    )
    return _tpu.reinterpret_cast(target, ref), reshaper.shape


_tc_lowering._reshape_memref = _reshape_memref_as_reinterpret

_orig_slice_memref = _tc_lowering._slice_memref


def _slice_memref_keep_layout(ref, indexer, ref_aval, ref_block_shape):
    ref_ty = _ir.MemRefType(ref.type)
    if "#tpu.tiled<(1" not in str(ref_ty):
        return _orig_slice_memref(ref, indexer, ref_aval, ref_block_shape)
    starts, sizes, strides, squeeze_dims, ref_block_shape = (
        _tc_lowering._indexer_to_start_size_stride(
            indexer, ref_block_shape, cast_to_index=False
        )
    )
    assert all((s is None or s == 1) for s in strides)
    assert not any(squeeze_dims)
    static_sizes = []
    dynamic_sizes = []
    for s in sizes:
        if not isinstance(s, _ir.Value):
            static_sizes.append(s)
        elif (v := _tc_lowering._fold_and_get_constant_value(s)) is not None:
            static_sizes.append(v)
        else:
            static_sizes.append(_ir.ShapedType.get_dynamic_size())
            dynamic_sizes.append(s)
    out_ty = _ir.MemRefType.get(
        static_sizes, ref_ty.element_type, ref_ty.layout, ref_ty.memory_space
    )
    return tuple(
        (_tpu.memref_slice(out_ty, ref, starts, dynamic_sizes), ref_block_shape)
    )


_tc_lowering._slice_memref = _slice_memref_keep_layout


def kernel(message, weight):
    (B,) = message.shape
    V, D = weight.shape
    info = plsc.get_sparse_core_info()
    nc, ns, L = info.num_cores, info.num_subcores, info.num_lanes
    nw = nc * ns
    bw = B // nw
    W = bw * D

    t_tiles = (V + 127) // 128
    v_full = (V // 128) * 128
    flat_words = D * V
    s_stride = t_tiles * 1024

    mesh = plsc.VectorSubcoreMesh(core_axis_name="c", subcore_axis_name="s")

    @functools.partial(
        pl.kernel,
        mesh=mesh,
        out_type=jax.ShapeDtypeStruct((D, B), jnp.float32),
        compiler_params=pltpu.CompilerParams(needs_layout_passes=False),
        scratch_types=[
            pltpu.VMEM((bw,), jnp.int32),
            pltpu.VMEM((W,), jnp.int32),
            pltpu.VMEM((256, 128), jnp.float32),
            pltpu.VMEM((8, V - v_full), jnp.float32),
            pltpu.SemaphoreType.DMA,
        ],
    )
    def gather_rows(
        idx_hbm, tab_hbm, out_hbm, bidx_v, addr_v, stage_v, tail_v, sem
    ):
        wid = lax.axis_index("s") * nc + lax.axis_index("c")
        base = wid * bw
        pltpu.sync_copy(idx_hbm.at[pl.ds(base, bw)], bidx_v)

        lanes = lax.iota(jnp.int32, L)
        n_t4 = bw // 128

        @pl.loop(0, bw // L, init_carry=jnp.int32(0), unroll=1)
        def build(q, maxv):
            vec = bidx_v[pl.ds(q * L, L)]
            addr0 = (vec >> 7) * 1024 + (vec & 127)
            qpos = (q >> 3) * 1024 + (q & 7) * L
            for s in range(2):
                for r in range(8):
                    a = addr0 + (s * s_stride + r * 128)
                    if s == 1:
                        a = jnp.where(a >= flat_words, 0, a)
                    addr_v[pl.ds(s * 4096 + r * 128 + qpos, L)] = a
            return jnp.maximum(maxv, jnp.max(vec))

        pltpu.async_copy(
            tab_hbm.reshape(flat_words, 1).at[addr_v],
            stage_v.at[pl.ds(0, 64), :].reshape(W, 1),
            sem,
        ).wait()


        @pl.when(build >= v_full)
        def _tail():
            pltpu.sync_copy(
                tab_hbm.at[pl.ds(8, 8), pl.ds(v_full, V - v_full)], tail_v
            )

            @pl.loop(0, bw // L, unroll=1)
            def patch(q):
                vec = bidx_v[pl.ds(q * L, L)]
                m = vec >= v_full
                col = jnp.where(m, vec - v_full, 0)
                qpos = 4096 + (q >> 3) * 1024 + (q & 7) * L + lanes
                for r in range(8):
                    val = plsc.load_gather(
                        tail_v, [jnp.full((L,), r, jnp.int32), col], mask=m
                    )
                    p = qpos + r * 128
                    plsc.store_scatter(stage_v, [p >> 7, p & 127], val, mask=m)

        pltpu.sync_copy(
            stage_v.at[pl.ds(0, 64), :].reshape(D, bw),
            out_hbm.at[:, pl.ds(base, bw)],
        )

    return gather_rows(message.astype(jnp.int32), weight.T).T
